# baseline (speedup 1.0000x reference)
import functools

import jax
import jax.numpy as jnp
from jax import lax
from jax.experimental import pallas as pl
from jax.experimental.pallas import tpu as pltpu
from jax.experimental.pallas import tpu_sc as plsc

NC = 2
NS = 16
NW = NC * NS

T = 16384
S = 50
D = 32
V = 1000000

TT = T // 128
TPW = TT // NW
NBLK = TPW * S
BPW = TPW * S * 128

_mesh = plsc.VectorSubcoreMesh(core_axis_name="c", subcore_axis_name="s")


@functools.partial(
    pl.kernel,
    mesh=_mesh,
    compiler_params=pltpu.CompilerParams(
        use_tc_tiling_on_sc=True, needs_layout_passes=False),
    out_type=jax.ShapeDtypeStruct((S, D // 8, TT, 8, 128), jnp.float32),
    scratch_types=[
        pltpu.VMEM((BPW,), jnp.int32),
        pltpu.VMEM((BPW,), jnp.int32),
        pltpu.VMEM((2, 128, 128), jnp.float32),
        pltpu.VMEM((2, D // 8, 8, 129), jnp.float32),
        pltpu.SemaphoreType.DMA,
        pltpu.SemaphoreType.DMA,
    ],
)
def _gather_kernel(widx_hbm, sub_hbm, table_hbm, out_hbm, widx_v, sub_v,
                   rows_v, stg_v, gsem, ssem):
    wid = lax.axis_index("s") * NC + lax.axis_index("c")
    gtt0 = wid * TPW
    pltpu.sync_copy(widx_hbm.at[pl.ds(wid * BPW, BPW)], widx_v)
    pltpu.sync_copy(sub_hbm.at[pl.ds(wid * BPW, BPW)], sub_v)
    iota = lax.iota(jnp.int32, 16)
    e4a = iota // 8
    e8v = iota - e4a * 8
    e4b = e4a + 2

    def blk_base(it):
        tl = it // S
        s = it - tl * S
        return tl, s, tl * (S * 128) + s * 128

    def start_gather(it, b):
        _, _, base = blk_base(it)
        pltpu.async_copy(table_hbm.at[widx_v.at[pl.ds(base, 128)]],
                         rows_v.at[b], gsem)

    def wait_gather(b):
        pltpu.make_async_copy(
            table_hbm.at[widx_v.at[pl.ds(0, 128)]], rows_v.at[b],
            gsem).wait()

    def start_store(it, b):
        tl, s, _ = blk_base(it)
        pltpu.async_copy(stg_v.at[b, :, :, pl.ds(0, 128)],
                         out_hbm.at[s, :, gtt0 + tl], ssem)

    def wait_store(b):
        pltpu.make_async_copy(stg_v.at[b, :, :, pl.ds(0, 128)],
                              out_hbm.at[0, :, gtt0], ssem).wait()

    def transpose_block(it, b):
        _, _, base = blk_base(it)
        rows = rows_v.at[b]
        stg = stg_v.at[b]

        def group(g, c):
            t0 = g * 16
            for k in range(16):
                tv = t0 + k + iota * 0
                sub = plsc.load_gather(sub_v, [base + tv]) + iota
                v0 = plsc.load_gather(rows, [tv, sub])
                v1 = plsc.load_gather(rows, [tv, sub + 16])
                plsc.store_scatter(stg, [e4a, e8v, tv], v0)
                plsc.store_scatter(stg, [e4b, e8v, tv], v1)
            return c

        lax.fori_loop(0, 8, group, 0)

    start_gather(0, 0)
    start_gather(1, 1)

    def body(it, carry):
        b = lax.rem(it, 2)
        wait_gather(b)

        @pl.when(it >= 2)
        def _():
            wait_store(b)

        transpose_block(it, b)
        start_store(it, b)

        @pl.when(it < NBLK - 2)
        def _():
            start_gather(it + 2, b)

        return carry

    lax.fori_loop(0, NBLK, body, 0)
    wait_store(0)
    wait_store(1)


def kernel(token_ids, weight):
    ids2 = (token_ids.astype(jnp.int32).T
            .reshape(S, TT, 128).transpose(1, 0, 2).reshape(TT, S * 128)
            .reshape(-1))
    widx = ids2 >> 2
    sub = (ids2 & 3) * 32
    wide = weight.reshape(V // 4, 4 * D)
    out5 = _gather_kernel(widx, sub, wide)
    return out5.transpose(2, 4, 0, 1, 3).reshape(T, S, D)


# speedup vs baseline: 1.6623x; 1.6623x over previous
import functools

import jax
import jax.numpy as jnp
from jax import lax
from jax.experimental import pallas as pl
from jax.experimental.pallas import tpu as pltpu
from jax.experimental.pallas import tpu_sc as plsc

NC = 2
NS = 16
NW = NC * NS

T = 16384
S = 50
D = 32
V = 1000000

TT = T // 128
TPW = TT // NW
NBLK = TPW * S

_mesh = plsc.VectorSubcoreMesh(core_axis_name="c", subcore_axis_name="s")


@functools.partial(
    pl.kernel,
    mesh=_mesh,
    compiler_params=pltpu.CompilerParams(
        use_tc_tiling_on_sc=False, needs_layout_passes=False),
    out_type=jax.ShapeDtypeStruct((S, D // 8, TT, 8, 128), jnp.float32),
    scratch_types=[
        pltpu.VMEM((TPW, S * 128), jnp.int32),
        pltpu.VMEM((4, 128, D), jnp.float32),
        pltpu.VMEM((4, D // 8, 8, 129), jnp.float32),
        pltpu.SemaphoreType.DMA,
        pltpu.SemaphoreType.DMA,
    ],
)
def _gather_kernel(ids_hbm, table_hbm, out_hbm, ids_v, rows_v, stg_v,
                   gsem, ssem):
    wid = lax.axis_index("s") * NC + lax.axis_index("c")
    gtt0 = wid * TPW
    pltpu.sync_copy(ids_hbm.at[pl.ds(gtt0, TPW)], ids_v)
    iota = lax.iota(jnp.int32, 16)
    e4a = iota // 8
    e8v = iota - e4a * 8
    e4b = e4a + 2

    def start_gather(it, b):
        tl = it // S
        s = it - tl * S
        pltpu.async_copy(
            table_hbm.at[ids_v.at[tl, pl.ds(s * 128, 128)]],
            rows_v.at[b], gsem)

    def wait_gather(b):
        pltpu.make_async_copy(
            table_hbm.at[ids_v.at[0, pl.ds(0, 128)]], rows_v.at[b],
            gsem).wait()

    def start_store(it, b):
        tl = it // S
        s = it - tl * S
        pltpu.async_copy(stg_v.at[b, :, :, pl.ds(0, 128)],
                         out_hbm.at[s, :, gtt0 + tl], ssem)

    def wait_store(b):
        pltpu.make_async_copy(stg_v.at[b, :, :, pl.ds(0, 128)],
                              out_hbm.at[0, :, gtt0], ssem).wait()

    def transpose_block(b):
        rows = rows_v.at[b]
        stg = stg_v.at[b]
        vals = {}
        for t in range(130):
            if t < 128:
                vals[t] = (rows[t, pl.ds(0, 16)], rows[t, pl.ds(16, 16)],
                           jnp.full((16,), t, jnp.int32))
            if t >= 2:
                v0, v1, tv = vals.pop(t - 2)
                plsc.store_scatter(stg, [e4a, e8v, tv], v0)
                plsc.store_scatter(stg, [e4b, e8v, tv], v1)

    for b in range(4):
        start_gather(b, b)

    def body(j, carry):
        for b in range(4):
            it = j * 4 + b
            wait_gather(b)

            @pl.when(it >= 4)
            def _():
                wait_store(b)

            transpose_block(b)
            start_store(it, b)

            @pl.when(it < NBLK - 4)
            def _():
                start_gather(it + 4, b)

        return carry

    lax.fori_loop(0, NBLK // 4, body, 0)
    for b in range(4):
        wait_store(b)


def kernel(token_ids, weight):
    ids2 = (token_ids.astype(jnp.int32).T
            .reshape(S, TT, 128).transpose(1, 0, 2).reshape(TT, S * 128))
    out5 = _gather_kernel(ids2, weight)
    return out5.transpose(2, 4, 0, 1, 3).reshape(T, S, D)
